# baseline (speedup 1.0000x reference)
import functools

import jax
import jax.numpy as jnp
from jax import lax
from jax.experimental import pallas as pl
from jax.experimental.pallas import tpu as pltpu
from jax.experimental.pallas import tpu_sc as plsc

_BATCH = 16384
_FEAT = 64
_LANES = 16

_NC = 2
_NS = 16
_NW = _NC * _NS
_ROWS_W = _BATCH // _NW
_IDX_CHUNK = 128
_N_CHUNKS = _ROWS_W // _IDX_CHUNK
_LBL_ROWS_W = _ROWS_W // _IDX_CHUNK
_GROUPS_Q = _IDX_CHUNK // _LANES

_PACK_BN = 16384
_HALF = _PACK_BN // 2


def _pack_pairs_body(in_ref, out_ref):
    x = in_ref[...]
    ta = jnp.transpose(x[:, :_HALF])
    tb = jnp.transpose(x[:, _HALF:])
    out_ref[...] = jnp.concatenate([ta, tb], axis=1)


def _pack_pairs(xt, n_rows):
    n = xt.shape[1]
    grid = (n + _PACK_BN - 1) // _PACK_BN
    return pl.pallas_call(
        _pack_pairs_body,
        grid=(grid,),
        in_specs=[pl.BlockSpec((_FEAT, _PACK_BN), lambda i: (0, i))],
        out_specs=pl.BlockSpec((_HALF, 128), lambda i: (i, 0)),
        out_shape=jax.ShapeDtypeStruct((grid * _HALF, 128), jnp.float32),
    )(xt)


def _pad_rows_body(in_ref, out_ref):
    t = jnp.transpose(in_ref[...])
    out_ref[...] = jnp.concatenate(
        [t, jnp.zeros((_PACK_BN, _FEAT), jnp.float32)], axis=1)


def _pad_rows(xt, n_rows):
    n = xt.shape[1]
    grid = (n + _PACK_BN - 1) // _PACK_BN
    return pl.pallas_call(
        _pad_rows_body,
        grid=(grid,),
        in_specs=[pl.BlockSpec((_FEAT, _PACK_BN), lambda i: (0, i))],
        out_specs=pl.BlockSpec((_PACK_BN, 128), lambda i: (i, 0)),
        out_shape=jax.ShapeDtypeStruct((n_rows, 128), jnp.float32),
    )(xt)


@functools.partial(
    pl.kernel,
    mesh=plsc.VectorSubcoreMesh(core_axis_name="c", subcore_axis_name="s"),
    compiler_params=pltpu.CompilerParams(use_tc_tiling_on_sc=True),
    out_type=jax.ShapeDtypeStruct((_NW, _LANES), jnp.float32),
    scratch_types=[
        pltpu.VMEM((_N_CHUNKS, _IDX_CHUNK), jnp.int32),
        pltpu.VMEM((_N_CHUNKS, _IDX_CHUNK), jnp.int32),
        pltpu.VMEM((_ROWS_W // _LANES, _LANES), jnp.int32),
        pltpu.VMEM((2, _IDX_CHUNK, 128), jnp.float32),
        pltpu.VMEM((2, _IDX_CHUNK, 128), jnp.float32),
        pltpu.VMEM((_LANES,), jnp.float32),
        pltpu.SemaphoreType.DMA,
        pltpu.SemaphoreType.DMA,
    ],
)
def _center_loss_partials(x_hbm, labels_hbm, centers_hbm, out_hbm,
                          idx_v, row_v, off_v, rows_v, x_v, acc_v,
                          sem_x, sem_g):
    wid = lax.axis_index("s") * _NC + lax.axis_index("c")

    pltpu.sync_copy(labels_hbm.at[pl.ds(wid * _LBL_ROWS_W, _LBL_ROWS_W)],
                    idx_v)
    bn_log = _PACK_BN.bit_length() - 1
    half_log = _HALF.bit_length() - 1
    for j in range(_N_CHUNKS):
        for k in range(_IDX_CHUNK // _LANES):
            v = idx_v[j, pl.ds(k * _LANES, _LANES)]
            blk = lax.shift_right_logical(v, jnp.int32(bn_log))
            row_v[j, pl.ds(k * _LANES, _LANES)] = (
                lax.shift_left(blk, jnp.int32(half_log))
                + jnp.bitwise_and(v, jnp.int32(_HALF - 1)))
            off_v[j * (_IDX_CHUNK // _LANES) + k, pl.ds(0, _LANES)] = (
                lax.shift_left(
                    jnp.bitwise_and(
                        lax.shift_right_logical(v, jnp.int32(half_log)),
                        jnp.int32(1)),
                    jnp.int32(6)))

    def start_quarter(q):
        slot = q % 2
        xc = pltpu.async_copy(
            x_hbm.at[pl.ds(wid * _ROWS_W + q * _IDX_CHUNK, _IDX_CHUNK)],
            x_v.at[slot], sem_x)
        gc = pltpu.async_copy(centers_hbm.at[row_v.at[q]], rows_v.at[slot],
                              sem_g)
        return (xc, gc)

    def compute_quarter(q, acc):
        slot = q % 2

        def group_body(g, acc):
            off16 = off_v[q * _GROUPS_Q + g, pl.ds(0, _LANES)]
            for i in range(_LANES):
                r = g * _LANES + i
                off = off16[i]
                for cth in range(4):
                    xv = x_v[slot, r, pl.ds(cth * _LANES, _LANES)]
                    cv = rows_v[slot, r, pl.ds(off + cth * _LANES, _LANES)]
                    d = xv - cv
                    acc = acc + d * d
            return acc
        return lax.fori_loop(0, _GROUPS_Q, group_body, acc)

    acc = jnp.zeros((_LANES,), jnp.float32)
    pend = start_quarter(0)
    for q in range(_N_CHUNKS):
        nxt = start_quarter(q + 1) if q + 1 < _N_CHUNKS else None
        for c in pend:
            c.wait()
        acc = compute_quarter(q, acc)
        pend = nxt

    acc_v[...] = acc
    pltpu.sync_copy(acc_v, out_hbm.at[wid])


def kernel(x, labels, centers):
    xp = _pad_rows(x.T, _BATCH)
    cp = _pack_pairs(centers.T, centers.shape[0])
    labels_r = labels.astype(jnp.int32).reshape(_BATCH // _IDX_CHUNK,
                                                _IDX_CHUNK)
    partials = _center_loss_partials(xp, labels_r, cp)
    return jnp.sum(partials) * (1.0 / _BATCH)


# speedup vs baseline: 1.0215x; 1.0215x over previous
import functools

import jax
import jax.numpy as jnp
from jax import lax
from jax.experimental import pallas as pl
from jax.experimental.pallas import tpu as pltpu
from jax.experimental.pallas import tpu_sc as plsc

_BATCH = 16384
_FEAT = 64
_LANES = 16

_NC = 2
_NS = 16
_NW = _NC * _NS
_ROWS_W = _BATCH // _NW
_IDX_CHUNK = 128
_N_CHUNKS = _ROWS_W // _IDX_CHUNK
_LBL_ROWS_W = _ROWS_W // _IDX_CHUNK
_GROUPS_Q = _IDX_CHUNK // _LANES

_PACK_BN = 8192
_HALF = _PACK_BN // 2


def _pack_pairs_body(in_ref, out_ref):
    x = in_ref[...]
    ta = jnp.transpose(x[:, :_HALF])
    tb = jnp.transpose(x[:, _HALF:])
    out_ref[...] = jnp.concatenate([ta, tb], axis=1)


def _pack_pairs(xt, n_rows):
    n = xt.shape[1]
    grid = (n + _PACK_BN - 1) // _PACK_BN
    return pl.pallas_call(
        _pack_pairs_body,
        grid=(grid,),
        in_specs=[pl.BlockSpec((_FEAT, _PACK_BN), lambda i: (0, i))],
        out_specs=pl.BlockSpec((_HALF, 128), lambda i: (i, 0)),
        out_shape=jax.ShapeDtypeStruct((grid * _HALF, 128), jnp.float32),
    )(xt)


def _pad_rows_body(in_ref, out_ref):
    t = jnp.transpose(in_ref[...])
    out_ref[...] = jnp.concatenate(
        [t, jnp.zeros((_PACK_BN, _FEAT), jnp.float32)], axis=1)


def _pad_rows(xt, n_rows):
    n = xt.shape[1]
    grid = (n + _PACK_BN - 1) // _PACK_BN
    return pl.pallas_call(
        _pad_rows_body,
        grid=(grid,),
        in_specs=[pl.BlockSpec((_FEAT, _PACK_BN), lambda i: (0, i))],
        out_specs=pl.BlockSpec((_PACK_BN, 128), lambda i: (i, 0)),
        out_shape=jax.ShapeDtypeStruct((n_rows, 128), jnp.float32),
    )(xt)


@functools.partial(
    pl.kernel,
    mesh=plsc.VectorSubcoreMesh(core_axis_name="c", subcore_axis_name="s"),
    compiler_params=pltpu.CompilerParams(use_tc_tiling_on_sc=True),
    out_type=jax.ShapeDtypeStruct((_NW, _LANES), jnp.float32),
    scratch_types=[
        pltpu.VMEM((_N_CHUNKS, _IDX_CHUNK), jnp.int32),
        pltpu.VMEM((_N_CHUNKS, _IDX_CHUNK), jnp.int32),
        pltpu.VMEM((_ROWS_W // _LANES, _LANES), jnp.int32),
        pltpu.VMEM((2, _IDX_CHUNK, 128), jnp.float32),
        pltpu.VMEM((2, _IDX_CHUNK, 128), jnp.float32),
        pltpu.VMEM((_LANES,), jnp.float32),
        pltpu.SemaphoreType.DMA,
        pltpu.SemaphoreType.DMA,
    ],
)
def _center_loss_partials(x_hbm, labels_hbm, centers_hbm, out_hbm,
                          idx_v, row_v, off_v, rows_v, x_v, acc_v,
                          sem_x, sem_g):
    wid = lax.axis_index("s") * _NC + lax.axis_index("c")

    pltpu.sync_copy(labels_hbm.at[pl.ds(wid * _LBL_ROWS_W, _LBL_ROWS_W)],
                    idx_v)
    bn_log = _PACK_BN.bit_length() - 1
    half_log = _HALF.bit_length() - 1
    for j in range(_N_CHUNKS):
        for k in range(_IDX_CHUNK // _LANES):
            v = idx_v[j, pl.ds(k * _LANES, _LANES)]
            blk = lax.shift_right_logical(v, jnp.int32(bn_log))
            row_v[j, pl.ds(k * _LANES, _LANES)] = (
                lax.shift_left(blk, jnp.int32(half_log))
                + jnp.bitwise_and(v, jnp.int32(_HALF - 1)))
            off_v[j * (_IDX_CHUNK // _LANES) + k, pl.ds(0, _LANES)] = (
                lax.shift_left(
                    jnp.bitwise_and(
                        lax.shift_right_logical(v, jnp.int32(half_log)),
                        jnp.int32(1)),
                    jnp.int32(6)))

    def start_quarter(q):
        slot = q % 2
        xc = pltpu.async_copy(
            x_hbm.at[pl.ds(wid * _ROWS_W + q * _IDX_CHUNK, _IDX_CHUNK)],
            x_v.at[slot], sem_x)
        gc = pltpu.async_copy(centers_hbm.at[row_v.at[q]], rows_v.at[slot],
                              sem_g)
        return (xc, gc)

    def compute_quarter(q, acc):
        slot = q % 2

        def group_body(g, acc):
            off16 = off_v[q * _GROUPS_Q + g, pl.ds(0, _LANES)]
            for i in range(_LANES):
                r = g * _LANES + i
                off = off16[i]
                for cth in range(4):
                    xv = x_v[slot, r, pl.ds(cth * _LANES, _LANES)]
                    cv = rows_v[slot, r, pl.ds(off + cth * _LANES, _LANES)]
                    d = xv - cv
                    acc = acc + d * d
            return acc
        return lax.fori_loop(0, _GROUPS_Q, group_body, acc)

    acc = jnp.zeros((_LANES,), jnp.float32)
    pend = start_quarter(0)
    for q in range(_N_CHUNKS):
        nxt = start_quarter(q + 1) if q + 1 < _N_CHUNKS else None
        for c in pend:
            c.wait()
        acc = compute_quarter(q, acc)
        pend = nxt

    acc_v[...] = acc
    pltpu.sync_copy(acc_v, out_hbm.at[wid])


def kernel(x, labels, centers):
    xp = _pad_rows(x.T, _BATCH)
    cp = _pack_pairs(centers.T, centers.shape[0])
    labels_r = labels.astype(jnp.int32).reshape(_BATCH // _IDX_CHUNK,
                                                _IDX_CHUNK)
    partials = _center_loss_partials(xp, labels_r, cp)
    return jnp.sum(partials) * (1.0 / _BATCH)
